# baseline (speedup 1.0000x reference)
import functools
import jax
import jax.numpy as jnp
from jax import lax
from jax.experimental import pallas as pl
from jax.experimental.pallas import tpu as pltpu
from jax.experimental.pallas import tpu_sc as plsc

EMBED = 2048
NLEV = 16
L = 16
NW = 32
B0 = 1024
B1 = 20
N = B0 * B1
BPW = B0 // NW
CHUNK = EMBED // L
UNR = 8
TBL = (NLEV - 1) * EMBED


def _encode_a_body(w_ref, t_ref, a_ref):
    w = w_ref[...]
    t = t_ref[...]
    ws = w[:-1, :]
    we = w[1:, :]
    a_ref[...] = jnp.where(ws == we, 1.5, t)


def _encode_s_body(w_ref, t_ref, s_ref):
    w = w_ref[...]
    ws = w[:-1, :]
    we = w[1:, :]
    flipf = jnp.where(ws == we, ws, -ws)
    s_ref[...] = jnp.where(flipf > 0.0, jnp.float32(-0.0), jnp.float32(0.0))


def _encode(weight, threshold):
    a = pl.pallas_call(
        _encode_a_body,
        out_shape=jax.ShapeDtypeStruct((NLEV - 1, EMBED), jnp.float32),
    )(weight, threshold)
    s = pl.pallas_call(
        _encode_s_body,
        out_shape=jax.ShapeDtypeStruct((NLEV - 1, EMBED), jnp.float32),
    )(weight, threshold)
    return a, s


def _sc_body(x_hbm, a_hbm, s_hbm, out_hbm, x_v, a_v, s_v, tau_v, gb_v, buf_v,
             sem0, sem1):
    cid = lax.axis_index("c")
    sid = lax.axis_index("s")
    wid = sid * 2 + cid
    base_b = wid * BPW
    pltpu.sync_copy(a_hbm, a_v.at[pl.ds(0, TBL)])
    pltpu.sync_copy(s_hbm, s_v.at[pl.ds(0, TBL)])
    pltpu.sync_copy(x_hbm.at[pl.ds(base_b * B1, BPW * B1)], x_v)
    lanes = lax.iota(jnp.int32, L)

    def pre(j, carry):
        xv = x_v[pl.ds(j * L, L)]
        span = jnp.clip(xv * float(NLEV - 1), 0.0, float(NLEV - 1))
        idxi = jnp.minimum(span.astype(jnp.int32), NLEV - 2)
        tau_v[pl.ds(j * L, L)] = span - idxi.astype(jnp.float32)
        gb_v[pl.ds(j * L, L)] = idxi * EMBED
        return carry

    lax.fori_loop(0, (BPW * B1) // L, pre, 0)

    a_wins = [a_v.at[pl.ds(k * L, TBL)] for k in range(UNR)]
    s_wins = [s_v.at[pl.ds(k * L, TBL)] for k in range(UNR)]

    def batch(g, carry):
        j = lax.div(g, 2)
        h = lax.rem(g, 2)
        slot = lax.rem(g, 2)
        @pl.when(g >= 2)
        def _wait():
            @pl.when(slot == 0)
            def _w0():
                pltpu.make_async_copy(buf_v.at[0], out_hbm.at[0, pl.ds(base_b, L)], sem0).wait()

            @pl.when(slot == 1)
            def _w1():
                pltpu.make_async_copy(buf_v.at[1], out_hbm.at[0, pl.ds(base_b, L)], sem1).wait()

        def row(r, carry2):
            rsplat = jnp.full((L,), (h * L + r) * B1 + j, jnp.int32)
            tau = plsc.load_gather(tau_v, [rsplat])
            ga0 = plsc.load_gather(gb_v, [rsplat]) + lanes

            def col(jc, ga):
                ts = [plsc.load_gather(a_wins[k], [ga]) for k in range(UNR)]
                ss = [plsc.load_gather(s_wins[k], [ga]) for k in range(UNR)]
                for k in range(UNR):
                    c = tau > ts[k]
                    z = jnp.where(c, 1.0, -1.0).astype(jnp.float32)
                    zi = plsc.bitcast(z, jnp.int32) ^ plsc.bitcast(ss[k], jnp.int32)
                    val = plsc.bitcast(zi, jnp.float32)
                    buf_v[slot, r, pl.ds(jc * (UNR * L) + k * L, L)] = val
                return ga + UNR * L

            lax.fori_loop(0, CHUNK // UNR, col, ga0)
            return carry2

        lax.fori_loop(0, L, row, 0)
        dst_row = base_b + h * L

        @pl.when(slot == 0)
        def _s0():
            pltpu.async_copy(buf_v.at[0], out_hbm.at[j, pl.ds(dst_row, L)], sem0)

        @pl.when(slot == 1)
        def _s1():
            pltpu.async_copy(buf_v.at[1], out_hbm.at[j, pl.ds(dst_row, L)], sem1)

        return carry

    lax.fori_loop(0, 2 * B1, batch, 0)
    pltpu.make_async_copy(buf_v.at[0], out_hbm.at[0, pl.ds(base_b, L)], sem0).wait()
    pltpu.make_async_copy(buf_v.at[1], out_hbm.at[0, pl.ds(base_b, L)], sem1).wait()


@jax.jit
def _run(x_flat, a_flat, s_flat):
    mesh = plsc.VectorSubcoreMesh(core_axis_name="c", subcore_axis_name="s")
    sc = pl.kernel(
        _sc_body,
        out_type=jax.ShapeDtypeStruct((B1, B0, EMBED), jnp.float32),
        mesh=mesh,
        compiler_params=pltpu.CompilerParams(
            needs_layout_passes=False,
            use_tc_tiling_on_sc=True,
        ),
        scratch_types=[
            pltpu.VMEM((BPW * B1,), jnp.float32),
            pltpu.VMEM((TBL + UNR * L,), jnp.float32),
            pltpu.VMEM((TBL + UNR * L,), jnp.float32),
            pltpu.VMEM((BPW * B1,), jnp.float32),
            pltpu.VMEM((BPW * B1,), jnp.int32),
            pltpu.VMEM((2, L, EMBED), jnp.float32),
            pltpu.SemaphoreType.DMA,
            pltpu.SemaphoreType.DMA,
        ],
    )
    return sc(x_flat, a_flat, s_flat)


def kernel(input, weight, threshold):
    a, s = _encode(weight, threshold)
    out_t = _run(input.reshape(N), a.reshape(TBL), s.reshape(TBL))
    return jnp.transpose(out_t, (1, 0, 2)).reshape(*input.shape, EMBED)


# speedup vs baseline: 1.0684x; 1.0684x over previous
import functools
import jax
import jax.numpy as jnp
from jax import lax
from jax.experimental import pallas as pl
from jax.experimental.pallas import tpu as pltpu
from jax.experimental.pallas import tpu_sc as plsc

EMBED = 2048
NLEV = 16
L = 16
NW = 32
B0 = 1024
B1 = 20
N = B0 * B1
BPW = B0 // NW
CHUNK = EMBED // L
UNR = 8
TBL = (NLEV - 1) * EMBED


def _encode_body(w_ref, t_ref, u_ref):
    w = w_ref[...]
    t = t_ref[...]
    ws = w[:-1, :]
    we = w[1:, :]
    u_ref[...] = ws * jnp.where(ws == we, -1.5, t)


def _encode(weight, threshold):
    return pl.pallas_call(
        _encode_body,
        out_shape=jax.ShapeDtypeStruct((NLEV - 1, EMBED), jnp.float32),
    )(weight, threshold)


def _sc_body(x_hbm, u_hbm, out_hbm, x_v, u_v, tau_v, gb_v, buf_v, sem0, sem1):
    cid = lax.axis_index("c")
    sid = lax.axis_index("s")
    wid = sid * 2 + cid
    base_b = wid * BPW
    pltpu.sync_copy(u_hbm, u_v)
    pltpu.sync_copy(x_hbm.at[pl.ds(base_b * B1, BPW * B1)], x_v)

    def pre(j, carry):
        xv = x_v[pl.ds(j * L, L)]
        span = jnp.clip(xv * float(NLEV - 1), 0.0, float(NLEV - 1))
        idxi = jnp.minimum(span.astype(jnp.int32), NLEV - 2)
        tau_v[pl.ds(j * L, L)] = span - idxi.astype(jnp.float32)
        gb_v[pl.ds(j * L, L)] = idxi * EMBED
        return carry

    lax.fori_loop(0, (BPW * B1) // L, pre, 0)

    def batch(g, carry):
        j = lax.div(g, 2)
        h = lax.rem(g, 2)
        slot = lax.rem(g, 2)
        @pl.when(g >= 2)
        def _wait():
            @pl.when(slot == 0)
            def _w0():
                pltpu.make_async_copy(buf_v.at[0], out_hbm.at[0, pl.ds(base_b, L)], sem0).wait()

            @pl.when(slot == 1)
            def _w1():
                pltpu.make_async_copy(buf_v.at[1], out_hbm.at[0, pl.ds(base_b, L)], sem1).wait()

        def row(r, carry2):
            rowidx = (h * L + r) * B1 + j
            tau = jnp.full((L,), tau_v[pl.ds(rowidx, L)][0], jnp.float32)
            gb = gb_v[pl.ds(rowidx, L)][0]

            def col(jc, sb):
                us = [u_v[pl.ds(sb + k * L, L)] for k in range(UNR)]
                for k in range(UNR):
                    u16 = us[k]
                    ub = plsc.bitcast(u16, jnp.int32)
                    neg = ub < 0
                    hit = tau > jnp.abs(u16)
                    val = jnp.where(neg != hit, 1.0, -1.0).astype(jnp.float32)
                    buf_v[slot, r, pl.ds(jc * (UNR * L) + k * L, L)] = val
                return sb + UNR * L

            lax.fori_loop(0, CHUNK // UNR, col, gb)
            return carry2

        lax.fori_loop(0, L, row, 0)
        dst_row = base_b + h * L

        @pl.when(slot == 0)
        def _s0():
            pltpu.async_copy(buf_v.at[0], out_hbm.at[j, pl.ds(dst_row, L)], sem0)

        @pl.when(slot == 1)
        def _s1():
            pltpu.async_copy(buf_v.at[1], out_hbm.at[j, pl.ds(dst_row, L)], sem1)

        return carry

    lax.fori_loop(0, 2 * B1, batch, 0)
    pltpu.make_async_copy(buf_v.at[0], out_hbm.at[0, pl.ds(base_b, L)], sem0).wait()
    pltpu.make_async_copy(buf_v.at[1], out_hbm.at[0, pl.ds(base_b, L)], sem1).wait()


@jax.jit
def _run(x_flat, u_flat):
    mesh = plsc.VectorSubcoreMesh(core_axis_name="c", subcore_axis_name="s")
    sc = pl.kernel(
        _sc_body,
        out_type=jax.ShapeDtypeStruct((B1, B0, EMBED), jnp.float32),
        mesh=mesh,
        compiler_params=pltpu.CompilerParams(
            needs_layout_passes=False,
            use_tc_tiling_on_sc=True,
        ),
        scratch_types=[
            pltpu.VMEM((BPW * B1,), jnp.float32),
            pltpu.VMEM((TBL,), jnp.float32),
            pltpu.VMEM((BPW * B1 + L,), jnp.float32),
            pltpu.VMEM((BPW * B1 + L,), jnp.int32),
            pltpu.VMEM((2, L, EMBED), jnp.float32),
            pltpu.SemaphoreType.DMA,
            pltpu.SemaphoreType.DMA,
        ],
    )
    return sc(x_flat, u_flat)


def kernel(input, weight, threshold):
    u = _encode(weight, threshold)
    out_t = _run(input.reshape(N), u.reshape(TBL))
    return jnp.transpose(out_t, (1, 0, 2)).reshape(*input.shape, EMBED)


# speedup vs baseline: 1.2295x; 1.1508x over previous
import functools
import jax
import jax.numpy as jnp
from jax import lax
from jax.experimental import pallas as pl
from jax.experimental.pallas import tpu as pltpu
from jax.experimental.pallas import tpu_sc as plsc

EMBED = 2048
NLEV = 16
L = 16
NW = 32
B0 = 1024
B1 = 20
N = B0 * B1
BPW = B0 // NW
CHUNK = EMBED // L
UNR = 8
TBL = (NLEV - 1) * EMBED


def _encode_body(w_ref, t_ref, u_ref):
    w = w_ref[...]
    t = t_ref[...]
    ws = w[:-1, :]
    we = w[1:, :]
    u_ref[...] = ws * jnp.where(ws == we, -1.5, t)


def _encode(weight, threshold):
    return pl.pallas_call(
        _encode_body,
        out_shape=jax.ShapeDtypeStruct((NLEV - 1, EMBED), jnp.float32),
    )(weight, threshold)


def _sc_body(x_hbm, u_hbm, out_hbm, x_v, u_v, tau_v, gb_v, buf_v, sem0, sem1):
    cid = lax.axis_index("c")
    sid = lax.axis_index("s")
    wid = sid * 2 + cid
    base_b = wid * BPW
    pltpu.sync_copy(u_hbm, u_v)
    pltpu.sync_copy(x_hbm.at[pl.ds(base_b * B1, BPW * B1)], x_v)

    def pre(j, carry):
        xv = x_v[pl.ds(j * L, L)]
        span = jnp.clip(xv * float(NLEV - 1), 0.0, float(NLEV - 1))
        idxi = jnp.minimum(span.astype(jnp.int32), NLEV - 2)
        tau_v[pl.ds(j * L, L)] = span - idxi.astype(jnp.float32)
        gb_v[pl.ds(j * L, L)] = idxi * EMBED
        return carry

    lax.fori_loop(0, (BPW * B1) // L, pre, 0)

    def batch(g, carry):
        j = lax.div(g, 2)
        h = lax.rem(g, 2)
        slot = lax.rem(g, 2)
        @pl.when(g >= 2)
        def _wait():
            @pl.when(slot == 0)
            def _w0():
                pltpu.make_async_copy(buf_v.at[0], out_hbm.at[0, pl.ds(base_b, L)], sem0).wait()

            @pl.when(slot == 1)
            def _w1():
                pltpu.make_async_copy(buf_v.at[1], out_hbm.at[0, pl.ds(base_b, L)], sem1).wait()

        def row(r, carry2):
            rowidx = (h * L + r) * B1 + j
            tau = jnp.full((L,), tau_v[pl.ds(rowidx, L)][0], jnp.float32)
            gb = gb_v[pl.ds(rowidx, L)][0]
            taub = plsc.bitcast(tau, jnp.int32)

            def col(jc, sb):
                us = [u_v[pl.ds(sb + k * L, L)] for k in range(UNR)]
                for k in range(UNR):
                    d = plsc.bitcast(us[k], jnp.int32) - taub
                    val = jnp.where(d < 0, 1.0, -1.0).astype(jnp.float32)
                    buf_v[slot, r, pl.ds(jc * (UNR * L) + k * L, L)] = val
                return sb + UNR * L

            lax.fori_loop(0, CHUNK // UNR, col, gb)
            return carry2

        lax.fori_loop(0, L, row, 0)
        dst_row = base_b + h * L

        @pl.when(slot == 0)
        def _s0():
            pltpu.async_copy(buf_v.at[0], out_hbm.at[j, pl.ds(dst_row, L)], sem0)

        @pl.when(slot == 1)
        def _s1():
            pltpu.async_copy(buf_v.at[1], out_hbm.at[j, pl.ds(dst_row, L)], sem1)

        return carry

    lax.fori_loop(0, 2 * B1, batch, 0)
    pltpu.make_async_copy(buf_v.at[0], out_hbm.at[0, pl.ds(base_b, L)], sem0).wait()
    pltpu.make_async_copy(buf_v.at[1], out_hbm.at[0, pl.ds(base_b, L)], sem1).wait()


@jax.jit
def _run(x_flat, u_flat):
    mesh = plsc.VectorSubcoreMesh(core_axis_name="c", subcore_axis_name="s")
    sc = pl.kernel(
        _sc_body,
        out_type=jax.ShapeDtypeStruct((B1, B0, EMBED), jnp.float32),
        mesh=mesh,
        compiler_params=pltpu.CompilerParams(
            needs_layout_passes=False,
            use_tc_tiling_on_sc=True,
        ),
        scratch_types=[
            pltpu.VMEM((BPW * B1,), jnp.float32),
            pltpu.VMEM((TBL,), jnp.float32),
            pltpu.VMEM((BPW * B1 + L,), jnp.float32),
            pltpu.VMEM((BPW * B1 + L,), jnp.int32),
            pltpu.VMEM((2, L, EMBED), jnp.float32),
            pltpu.SemaphoreType.DMA,
            pltpu.SemaphoreType.DMA,
        ],
    )
    return sc(x_flat, u_flat)


def kernel(input, weight, threshold):
    u = _encode(weight, threshold)
    out_t = _run(input.reshape(N), u.reshape(TBL))
    return jnp.transpose(out_t, (1, 0, 2)).reshape(*input.shape, EMBED)


# speedup vs baseline: 1.4380x; 1.1696x over previous
import functools
import jax
import jax.numpy as jnp
from jax import lax
from jax.experimental import pallas as pl
from jax.experimental.pallas import tpu as pltpu
from jax.experimental.pallas import tpu_sc as plsc

EMBED = 2048
NLEV = 16
L = 16
NW = 32
B0 = 1024
B1 = 20
N = B0 * B1
BPW = B0 // NW
CHUNK = EMBED // L
UNR = 16
TBL = (NLEV - 1) * EMBED


def _encode_body(w_ref, t_ref, u_ref):
    w = w_ref[...]
    t = t_ref[...]
    ws = w[:-1, :]
    we = w[1:, :]
    u_ref[...] = ws * jnp.where(ws == we, -1.5, t)


def _encode(weight, threshold):
    return pl.pallas_call(
        _encode_body,
        out_shape=jax.ShapeDtypeStruct((NLEV - 1, EMBED), jnp.float32),
    )(weight, threshold)


def _sc_body(x_hbm, u_hbm, out_hbm, x_v, u_v, tau_v, gb_v, buf_v, sem0, sem1):
    cid = lax.axis_index("c")
    sid = lax.axis_index("s")
    wid = sid * 2 + cid
    base_b = wid * BPW
    pltpu.sync_copy(u_hbm, u_v)
    pltpu.sync_copy(x_hbm.at[pl.ds(base_b * B1, BPW * B1)], x_v)

    def pre(j, carry):
        xv = x_v[pl.ds(j * L, L)]
        span = jnp.clip(xv * float(NLEV - 1), 0.0, float(NLEV - 1))
        idxi = jnp.minimum(span.astype(jnp.int32), NLEV - 2)
        tau_v[pl.ds(j * L, L)] = span - idxi.astype(jnp.float32)
        gb_v[pl.ds(j * L, L)] = idxi * EMBED
        return carry

    lax.fori_loop(0, (BPW * B1) // L, pre, 0)

    def batch(g, carry):
        j = lax.div(g, 2)
        h = lax.rem(g, 2)
        slot = lax.rem(g, 2)
        @pl.when(g >= 2)
        def _wait():
            @pl.when(slot == 0)
            def _w0():
                pltpu.make_async_copy(buf_v.at[0], out_hbm.at[0, pl.ds(base_b, L)], sem0).wait()

            @pl.when(slot == 1)
            def _w1():
                pltpu.make_async_copy(buf_v.at[1], out_hbm.at[0, pl.ds(base_b, L)], sem1).wait()

        def row(r, carry2):
            rowidx = (h * L + r) * B1 + j
            tau = jnp.full((L,), tau_v[pl.ds(rowidx, L)][0], jnp.float32)
            gb = gb_v[pl.ds(rowidx, L)][0]
            taub = plsc.bitcast(tau, jnp.int32)

            def col(jc, sb):
                us = [u_v[pl.ds(sb + k * L, L)] for k in range(UNR)]
                for k in range(UNR):
                    d = plsc.bitcast(us[k], jnp.int32) - taub
                    val = jnp.where(d < 0, 1.0, -1.0).astype(jnp.float32)
                    buf_v[slot, r, pl.ds(jc * (UNR * L) + k * L, L)] = val
                return sb + UNR * L

            lax.fori_loop(0, CHUNK // UNR, col, gb)
            return carry2

        lax.fori_loop(0, L, row, 0)
        dst_row = base_b + h * L

        @pl.when(slot == 0)
        def _s0():
            pltpu.async_copy(buf_v.at[0], out_hbm.at[j, pl.ds(dst_row, L)], sem0)

        @pl.when(slot == 1)
        def _s1():
            pltpu.async_copy(buf_v.at[1], out_hbm.at[j, pl.ds(dst_row, L)], sem1)

        return carry

    lax.fori_loop(0, 2 * B1, batch, 0)
    pltpu.make_async_copy(buf_v.at[0], out_hbm.at[0, pl.ds(base_b, L)], sem0).wait()
    pltpu.make_async_copy(buf_v.at[1], out_hbm.at[0, pl.ds(base_b, L)], sem1).wait()


@jax.jit
def _run(x_flat, u_flat):
    mesh = plsc.VectorSubcoreMesh(core_axis_name="c", subcore_axis_name="s")
    sc = pl.kernel(
        _sc_body,
        out_type=jax.ShapeDtypeStruct((B1, B0, EMBED), jnp.float32),
        mesh=mesh,
        compiler_params=pltpu.CompilerParams(
            needs_layout_passes=False,
            use_tc_tiling_on_sc=True,
        ),
        scratch_types=[
            pltpu.VMEM((BPW * B1,), jnp.float32),
            pltpu.VMEM((TBL,), jnp.float32),
            pltpu.VMEM((BPW * B1 + L,), jnp.float32),
            pltpu.VMEM((BPW * B1 + L,), jnp.int32),
            pltpu.VMEM((2, L, EMBED), jnp.float32),
            pltpu.SemaphoreType.DMA,
            pltpu.SemaphoreType.DMA,
        ],
    )
    return sc(x_flat, u_flat)


def kernel(input, weight, threshold):
    u = _encode(weight, threshold)
    out_t = _run(input.reshape(N), u.reshape(TBL))
    return jnp.transpose(out_t, (1, 0, 2)).reshape(*input.shape, EMBED)
